# baseline (speedup 1.0000x reference)
import jax
import jax.numpy as jnp
from jax.experimental import pallas as pl
from jax.experimental.pallas import tpu as pltpu

N = 5000
D = 4096
C = 81
CW = C + 4
CH = 200
NCH = N // CH
NBUF = 8


def _head_kernel(f_hbm, p_ref, w_ref, b_ref, cls_ref, box_ref, buf, sems):
    w = w_ref[...]
    bvec = b_ref[...]

    def start(chunk, slot):
        pltpu.make_async_copy(
            f_hbm.at[pl.ds(chunk * CH, CH), :], buf.at[slot],
            sems.at[slot]).start()

    for i in range(NBUF - 1):
        start(i, i)
    for i in range(NCH):
        slot = i % NBUF
        nxt = i + NBUF - 1
        if nxt < NCH:
            start(nxt, nxt % NBUF)
        pltpu.make_async_copy(
            f_hbm.at[pl.ds(i * CH, CH), :], buf.at[slot],
            sems.at[slot]).wait()
        acc = jnp.dot(buf[slot], w, preferred_element_type=jnp.float32)
        acc = acc + bvec
        cls_ref[pl.ds(i * CH, CH), :] = acc[:, :C]

        d = acc[:, C:CW]
        p = p_ref[pl.ds(i * CH, CH), :]
        px, py, pw, ph = p[:, 0:1], p[:, 1:2], p[:, 2:3], p[:, 3:4]
        x = d[:, 0:1] * pw + px
        y = d[:, 1:2] * ph + py
        ew = jnp.exp(d[:, 2:3])
        box_ref[pl.ds(i * CH, CH), :] = jnp.concatenate(
            [x, y, ew * pw, ew * ph], axis=1)


def kernel(feats, proposals_xywh, W_cls, b_cls, W_reg, b_reg):
    w_t = jnp.concatenate([W_cls, W_reg], axis=0).T
    b = jnp.concatenate([b_cls, b_reg]).reshape(1, CW)
    cls_out, box_out = pl.pallas_call(
        _head_kernel,
        in_specs=[
            pl.BlockSpec(memory_space=pltpu.MemorySpace.HBM),
            pl.BlockSpec(memory_space=pltpu.MemorySpace.VMEM),
            pl.BlockSpec(memory_space=pltpu.MemorySpace.VMEM),
            pl.BlockSpec(memory_space=pltpu.MemorySpace.VMEM),
        ],
        out_specs=[
            pl.BlockSpec(memory_space=pltpu.MemorySpace.VMEM),
            pl.BlockSpec(memory_space=pltpu.MemorySpace.VMEM),
        ],
        out_shape=[
            jax.ShapeDtypeStruct((N, C), jnp.float32),
            jax.ShapeDtypeStruct((N, 4), jnp.float32),
        ],
        scratch_shapes=[
            pltpu.VMEM((NBUF, CH, D), jnp.float32),
            pltpu.SemaphoreType.DMA((NBUF,)),
        ],
    )(feats, proposals_xywh, w_t, b)
    return (cls_out, box_out)


# speedup vs baseline: 1.0694x; 1.0694x over previous
import jax
import jax.numpy as jnp
from jax.experimental import pallas as pl
from jax.experimental.pallas import tpu as pltpu

N = 5000
D = 4096
C = 81
CW = C + 4
CH = 200
NCH = N // CH
NBUF = 8


def _head_kernel(f_hbm, p_ref, w_ref, b_ref, cls_ref, box_ref, buf, sems):
    w = w_ref[...].astype(jnp.bfloat16)
    bvec = b_ref[...]

    def start(chunk, slot):
        pltpu.make_async_copy(
            f_hbm.at[pl.ds(chunk * CH, CH), :], buf.at[slot],
            sems.at[slot]).start()

    for i in range(NBUF - 1):
        start(i, i)
    for i in range(NCH):
        slot = i % NBUF
        nxt = i + NBUF - 1
        if nxt < NCH:
            start(nxt, nxt % NBUF)
        pltpu.make_async_copy(
            f_hbm.at[pl.ds(i * CH, CH), :], buf.at[slot],
            sems.at[slot]).wait()
        acc = jnp.dot(buf[slot].astype(jnp.bfloat16), w,
                      preferred_element_type=jnp.float32)
        acc = acc + bvec
        cls_ref[pl.ds(i * CH, CH), :] = acc[:, :C]

        d = acc[:, C:CW]
        p = p_ref[pl.ds(i * CH, CH), :]
        px, py, pw, ph = p[:, 0:1], p[:, 1:2], p[:, 2:3], p[:, 3:4]
        x = d[:, 0:1] * pw + px
        y = d[:, 1:2] * ph + py
        ew = jnp.exp(d[:, 2:3])
        box_ref[pl.ds(i * CH, CH), :] = jnp.concatenate(
            [x, y, ew * pw, ew * ph], axis=1)


def kernel(feats, proposals_xywh, W_cls, b_cls, W_reg, b_reg):
    w_t = jnp.concatenate([W_cls, W_reg], axis=0).T
    b = jnp.concatenate([b_cls, b_reg]).reshape(1, CW)
    cls_out, box_out = pl.pallas_call(
        _head_kernel,
        in_specs=[
            pl.BlockSpec(memory_space=pltpu.MemorySpace.HBM),
            pl.BlockSpec(memory_space=pltpu.MemorySpace.VMEM),
            pl.BlockSpec(memory_space=pltpu.MemorySpace.VMEM),
            pl.BlockSpec(memory_space=pltpu.MemorySpace.VMEM),
        ],
        out_specs=[
            pl.BlockSpec(memory_space=pltpu.MemorySpace.VMEM),
            pl.BlockSpec(memory_space=pltpu.MemorySpace.VMEM),
        ],
        out_shape=[
            jax.ShapeDtypeStruct((N, C), jnp.float32),
            jax.ShapeDtypeStruct((N, 4), jnp.float32),
        ],
        scratch_shapes=[
            pltpu.VMEM((NBUF, CH, D), jnp.float32),
            pltpu.SemaphoreType.DMA((NBUF,)),
        ],
    )(feats, proposals_xywh, w_t, b)
    return (cls_out, box_out)
